# baseline (speedup 1.0000x reference)
import functools

import jax
import jax.numpy as jnp
from jax import lax
from jax.experimental import pallas as pl
from jax.experimental.pallas import tpu as pltpu
from jax.experimental.pallas import tpu_sc as plsc

N = 10000
E = 320000
D = 128
FP = 2048
G = 512

NC = 2
NS = 16
NW = NC * NS

K = 80
CH = 128
E_PAD = NW * CH * K
GC = 8
NACC = N + 8
NBUF = 4
ROWS_MAIN = 624
ROWS_TAIL = N - NS * ROWS_MAIN
ZTAIL = NACC - NS * ROWS_MAIN


def _sc_agg_body(table, src3d, dst3d, out, acc, src_g, dst_g, buf0, buf1,
                 buf2, buf3, g0s, g1s, g2s, g3s, s0s, s1s, s2s, s3s):
    c = lax.axis_index("c")
    s = lax.axis_index("s")
    wid = c * NS + s
    bufs = (buf0, buf1, buf2, buf3)
    gsems = (g0s, g1s, g2s, g3s)
    ssems = (s0s, s1s, s2s, s3s)

    zero = jnp.zeros((16,), jnp.float32)

    def zrow(r, carry):
        for cc in range(D // 16):
            buf0[r, pl.ds(cc * 16, 16)] = zero
        return carry

    lax.fori_loop(0, K, zrow, 0)
    base_row = s * ROWS_MAIN
    for kk in range(ROWS_MAIN // K):
        pltpu.sync_copy(buf0, acc.at[pl.ds(base_row + kk * K, K)])
    rem = ROWS_MAIN % K
    if rem:
        pltpu.sync_copy(
            buf0.at[pl.ds(0, rem)],
            acc.at[pl.ds(base_row + (ROWS_MAIN // K) * K, rem)])

    @pl.when(s == NS - 1)
    def _():
        pltpu.sync_copy(buf0.at[pl.ds(0, ZTAIL)],
                        acc.at[pl.ds(NS * ROWS_MAIN, ZTAIL)])

    plsc.subcore_barrier()

    pltpu.sync_copy(src3d.at[wid, pl.ds(0, GC)], src_g.at[0])
    pltpu.sync_copy(dst3d.at[wid, pl.ds(0, GC)], dst_g.at[0])
    pltpu.async_copy(table.at[src_g.at[0, 0]], buf0, g0s)
    pltpu.async_copy(table.at[src_g.at[0, 1]], buf1, g1s)
    pltpu.async_copy(table.at[src_g.at[0, 2]], buf2, g2s)

    def step16(kk, carry):
        j0 = 16 * kk
        for b in range(16):
            j = j0 + b
            rb = b % 4
            bn = (b + 3) % 4
            s_j = (b // GC) % 2
            r_j = b % GC
            s_n = ((b + 3) // GC) % 2
            r_n = (b + 3) % GC

            if b == 0:
                @pl.when(j >= 1)
                def _():
                    pltpu.make_async_copy(
                        bufs[bn], acc.at[dst_g.at[0, 0]], ssems[bn]).wait()
            else:
                pltpu.make_async_copy(
                    bufs[bn], acc.at[dst_g.at[0, 0]], ssems[bn]).wait()

            if b == 5:
                gr0 = pl.multiple_of((2 * kk + 1) * GC, GC)
                pltpu.sync_copy(src3d.at[wid, pl.ds(gr0, GC)], src_g.at[1])
                pltpu.sync_copy(dst3d.at[wid, pl.ds(gr0, GC)], dst_g.at[1])
            if b == 13:
                @pl.when(2 * kk + 2 < CH // GC)
                def _():
                    gr1 = pl.multiple_of((2 * kk + 2) * GC, GC)
                    pltpu.sync_copy(src3d.at[wid, pl.ds(gr1, GC)],
                                    src_g.at[0])
                    pltpu.sync_copy(dst3d.at[wid, pl.ds(gr1, GC)],
                                    dst_g.at[0])

            @pl.when(j + 3 < CH)
            def _():
                pltpu.async_copy(table.at[src_g.at[s_n, r_n]], bufs[bn],
                                 gsems[bn])

            pltpu.make_async_copy(table.at[src_g.at[0, 0]], bufs[rb],
                                  gsems[rb]).wait()
            pltpu.async_copy(bufs[rb], acc.at[dst_g.at[s_j, r_j]],
                             ssems[rb], add=True)
        return carry

    lax.fori_loop(0, CH // 16, step16, 0)
    pltpu.make_async_copy(bufs[3], acc.at[dst_g.at[0, 0]], ssems[3]).wait()

    plsc.subcore_barrier()
    pltpu.sync_copy(acc.at[pl.ds(base_row, ROWS_MAIN)],
                    out.at[c, pl.ds(base_row, ROWS_MAIN)])

    @pl.when(s == NS - 1)
    def _():
        pltpu.sync_copy(acc.at[pl.ds(NS * ROWS_MAIN, ROWS_TAIL)],
                        out.at[c, pl.ds(NS * ROWS_MAIN, ROWS_TAIL)])


_sc_agg = functools.partial(
    pl.kernel,
    out_type=jax.ShapeDtypeStruct((NC, N, D), jnp.float32),
    mesh=plsc.VectorSubcoreMesh(core_axis_name="c", subcore_axis_name="s",
                                num_cores=NC, num_subcores=NS),
    scratch_types=[
        pltpu.VMEM_SHARED((NACC, D), jnp.float32),
        pltpu.VMEM((2, GC, K), jnp.int32),
        pltpu.VMEM((2, GC, K), jnp.int32),
        pltpu.VMEM((K, D), jnp.float32),
        pltpu.VMEM((K, D), jnp.float32),
        pltpu.VMEM((K, D), jnp.float32),
        pltpu.VMEM((K, D), jnp.float32),
        pltpu.SemaphoreType.DMA,
        pltpu.SemaphoreType.DMA,
        pltpu.SemaphoreType.DMA,
        pltpu.SemaphoreType.DMA,
        pltpu.SemaphoreType.DMA,
        pltpu.SemaphoreType.DMA,
        pltpu.SemaphoreType.DMA,
        pltpu.SemaphoreType.DMA,
    ],
)(_sc_agg_body)


def _tc_layer_body(p_ref, x_ref, w_ref, b_ref, o_ref):
    sm = p_ref[0] + p_ref[1] + x_ref[...]
    z = jnp.dot(sm, w_ref[...], preferred_element_type=jnp.float32)
    o_ref[...] = jax.nn.sigmoid(z + b_ref[...])


_LAYER_BLK = 2000


def _tc_layer(p, x, wt, b):
    nb = N // _LAYER_BLK
    return pl.pallas_call(
        _tc_layer_body,
        grid=(nb,),
        in_specs=[
            pl.BlockSpec((NC, _LAYER_BLK, D), lambda i: (0, i, 0)),
            pl.BlockSpec((_LAYER_BLK, D), lambda i: (i, 0)),
            pl.BlockSpec((D, D), lambda i: (0, 0)),
            pl.BlockSpec((1, D), lambda i: (0, 0)),
        ],
        out_specs=pl.BlockSpec((_LAYER_BLK, D), lambda i: (i, 0)),
        out_shape=jax.ShapeDtypeStruct((N, D), jnp.float32),
    )(p, x, wt, b)


_FIN_BLK = 400


def _fp_contrib(h_ref, w_ref, b_ref, bt_ref):
    logits = jnp.dot(h_ref[...].astype(jnp.bfloat16), w_ref[...],
                     preferred_element_type=jnp.float32) + b_ref[...]
    e = jnp.exp(logits)
    fp = (e / jnp.sum(e, axis=1, keepdims=True)).astype(jnp.bfloat16)
    gid = bt_ref[0, 0, :]
    onehot = (gid[:, None] == lax.broadcasted_iota(
        jnp.int32, (_FIN_BLK, G), 1)).astype(jnp.bfloat16)
    return lax.dot_general(onehot, fp, (((0,), (0,)), ((), ())),
                           preferred_element_type=jnp.float32)


def _tc_fp1_body(h_ref, w_ref, b_ref, bt_ref, o_ref):
    i = pl.program_id(0)
    contrib = _fp_contrib(h_ref, w_ref, b_ref, bt_ref)

    @pl.when(i == 0)
    def _():
        o_ref[...] = contrib

    @pl.when(i > 0)
    def _():
        o_ref[...] += contrib


def _tc_fp2_body(acc_ref, h_ref, w_ref, b_ref, bt_ref, o_ref):
    i = pl.program_id(0)
    contrib = _fp_contrib(h_ref, w_ref, b_ref, bt_ref)

    @pl.when(i == 0)
    def _():
        o_ref[...] = acc_ref[...] + contrib

    @pl.when(i > 0)
    def _():
        o_ref[...] += contrib


_FIN_SPECS = [
    pl.BlockSpec((_FIN_BLK, D), lambda i: (i, 0)),
    pl.BlockSpec((D, FP), lambda i: (0, 0)),
    pl.BlockSpec((1, FP), lambda i: (0, 0)),
    pl.BlockSpec((1, 1, _FIN_BLK), lambda i: (i, 0, 0)),
]


def _tc_fp1(h, wt, b, batch3d):
    return pl.pallas_call(
        _tc_fp1_body,
        grid=(N // _FIN_BLK,),
        in_specs=_FIN_SPECS,
        out_specs=pl.BlockSpec((G, FP), lambda i: (0, 0)),
        out_shape=jax.ShapeDtypeStruct((G, FP), jnp.float32),
    )(h, wt, b, batch3d)


def _tc_fp2(acc, h, wt, b, batch3d):
    return pl.pallas_call(
        _tc_fp2_body,
        grid=(N // _FIN_BLK,),
        in_specs=[pl.BlockSpec((G, FP), lambda i: (0, 0))] + _FIN_SPECS,
        out_specs=pl.BlockSpec((G, FP), lambda i: (0, 0)),
        out_shape=jax.ShapeDtypeStruct((G, FP), jnp.float32),
    )(acc, h, wt, b, batch3d)


def kernel(x, edge_index, batch, H1_w, H1_b, W1_w, W1_b, H2_w, H2_b, W2_w,
           W2_b):
    npad = E_PAD - E
    src3d = jnp.concatenate(
        [edge_index[0], jnp.zeros((npad,), jnp.int32)]).reshape(NW, CH, K)
    dst3d = jnp.concatenate(
        [edge_index[1],
         N + (jnp.arange(npad, dtype=jnp.int32) % 8)]).reshape(NW, CH, K)
    batch3d = batch.reshape(N // _FIN_BLK, 1, _FIN_BLK)
    w1t = W1_w.T.astype(jnp.bfloat16)
    w2t = W2_w.T.astype(jnp.bfloat16)

    p1 = _sc_agg(x, src3d, dst3d)
    h1 = _tc_layer(p1, x, H1_w.T, H1_b.reshape(1, D))
    p2 = _sc_agg(h1, src3d, dst3d)
    acc1 = _tc_fp1(h1, w1t, W1_b.reshape(1, FP), batch3d)
    h2 = _tc_layer(p2, h1, H2_w.T, H2_b.reshape(1, D))
    return _tc_fp2(acc1, h2, w2t, W2_b.reshape(1, FP), batch3d)


# speedup vs baseline: 3.0791x; 3.0791x over previous
import functools

import jax
import jax.numpy as jnp
from jax import lax
from jax.experimental import pallas as pl
from jax.experimental.pallas import tpu as pltpu
from jax.experimental.pallas import tpu_sc as plsc

N = 10000
E = 320000
D = 128
FP = 2048
G = 512

NC = 2
NS = 16
NW = NC * NS

K = 125
PER_TILE = E // NW
CH = PER_TILE // K
GC = 16
NG = CH // GC
ROWS_MAIN = 624
ROWS_TAIL = N - NS * ROWS_MAIN


def _sc_agg_body(table, src2d, dst2d, out, acc, src_g, dst_g, buf0, buf1,
                 g0s, g1s, i0s, i1s):
    c = lax.axis_index("c")
    s = lax.axis_index("s")
    wid = c * NS + s
    row0 = wid * CH
    bufs = (buf0, buf1)
    gsems = (g0s, g1s)
    isems = (i0s, i1s)

    pltpu.async_copy(src2d.at[pl.ds(row0, GC)], src_g.at[0], i0s)
    pltpu.async_copy(dst2d.at[pl.ds(row0, GC)], dst_g.at[0], i0s)
    pltpu.make_async_copy(src2d.at[pl.ds(row0, GC)], src_g.at[0],
                          i0s).wait()
    pltpu.make_async_copy(dst2d.at[pl.ds(row0, GC)], dst_g.at[0],
                          i0s).wait()
    pltpu.async_copy(table.at[src_g.at[0, 0]], buf0, g0s)
    pltpu.async_copy(src2d.at[pl.ds(row0 + GC, GC)], src_g.at[1], i1s)
    pltpu.async_copy(dst2d.at[pl.ds(row0 + GC, GC)], dst_g.at[1], i1s)

    zero = jnp.zeros((16,), jnp.float32)

    def zrow(r, carry):
        for cc in range(D // 16):
            buf1[r, pl.ds(cc * 16, 16)] = zero
        return carry

    lax.fori_loop(0, K, zrow, 0)
    base_row = s * ROWS_MAIN
    for kk in range(ROWS_MAIN // K):
        pltpu.sync_copy(buf1, acc.at[pl.ds(base_row + kk * K, K)])
    rem = ROWS_MAIN % K
    if rem:
        pltpu.sync_copy(
            buf1.at[pl.ds(0, rem)],
            acc.at[pl.ds(base_row + (ROWS_MAIN // K) * K, rem)])

    @pl.when(s == NS - 1)
    def _():
        pltpu.sync_copy(buf1.at[pl.ds(0, ROWS_TAIL)],
                        acc.at[pl.ds(NS * ROWS_MAIN, ROWS_TAIL)])

    plsc.subcore_barrier()
    pltpu.async_copy(table.at[src_g.at[0, 1]], buf1, g1s)

    for j in range(CH):
        b = j % 2
        slot = (j // GC) % 2
        row = j % GC
        pltpu.make_async_copy(table.at[src_g.at[0, 0]], bufs[b],
                              gsems[b]).wait()
        pltpu.sync_copy(bufs[b], acc.at[dst_g.at[slot, row]], add=True)
        jn = j + 2
        if jn < CH:
            slot_n = (jn // GC) % 2
            row_n = jn % GC
            if row_n == 0:
                pltpu.make_async_copy(src2d.at[pl.ds(row0, GC)],
                                      src_g.at[slot_n], isems[slot_n]).wait()
                pltpu.make_async_copy(dst2d.at[pl.ds(row0, GC)],
                                      dst_g.at[slot_n], isems[slot_n]).wait()
            pltpu.async_copy(table.at[src_g.at[slot_n, row_n]], bufs[b],
                             gsems[b])
        if row == GC - 1 and (j // GC) + 2 < NG:
            gbase = row0 + ((j // GC) + 2) * GC
            pltpu.async_copy(src2d.at[pl.ds(gbase, GC)], src_g.at[slot],
                             isems[slot])
            pltpu.async_copy(dst2d.at[pl.ds(gbase, GC)], dst_g.at[slot],
                             isems[slot])

    plsc.subcore_barrier()
    pltpu.sync_copy(acc.at[pl.ds(base_row, ROWS_MAIN)],
                    out.at[c, pl.ds(base_row, ROWS_MAIN)])

    @pl.when(s == NS - 1)
    def _():
        pltpu.sync_copy(acc.at[pl.ds(NS * ROWS_MAIN, ROWS_TAIL)],
                        out.at[c, pl.ds(NS * ROWS_MAIN, ROWS_TAIL)])


_sc_agg = functools.partial(
    pl.kernel,
    out_type=jax.ShapeDtypeStruct((NC, N, D), jnp.float32),
    mesh=plsc.VectorSubcoreMesh(core_axis_name="c", subcore_axis_name="s",
                                num_cores=NC, num_subcores=NS),
    scratch_types=[
        pltpu.VMEM_SHARED((N, D), jnp.float32),
        pltpu.VMEM((2, GC, K), jnp.int32),
        pltpu.VMEM((2, GC, K), jnp.int32),
        pltpu.VMEM((K, D), jnp.float32),
        pltpu.VMEM((K, D), jnp.float32),
        pltpu.SemaphoreType.DMA,
        pltpu.SemaphoreType.DMA,
        pltpu.SemaphoreType.DMA,
        pltpu.SemaphoreType.DMA,
    ],
)(_sc_agg_body)


def _tc_layer_body(p_ref, x_ref, w_ref, b_ref, o_ref):
    sm = p_ref[0] + p_ref[1] + x_ref[...]
    z = jnp.dot(sm, w_ref[...], preferred_element_type=jnp.float32)
    o_ref[...] = jax.nn.sigmoid(z + b_ref[...])


_LAYER_BLK = 2000


def _tc_layer(p, x, wt, b):
    nb = N // _LAYER_BLK
    return pl.pallas_call(
        _tc_layer_body,
        grid=(nb,),
        in_specs=[
            pl.BlockSpec((NC, _LAYER_BLK, D), lambda i: (0, i, 0)),
            pl.BlockSpec((_LAYER_BLK, D), lambda i: (i, 0)),
            pl.BlockSpec((D, D), lambda i: (0, 0)),
            pl.BlockSpec((1, D), lambda i: (0, 0)),
        ],
        out_specs=pl.BlockSpec((_LAYER_BLK, D), lambda i: (i, 0)),
        out_shape=jax.ShapeDtypeStruct((N, D), jnp.float32),
    )(p, x, wt, b)


_FIN_BLK = 400


def _fp_contrib(h_ref, w_ref, b_ref, bt_ref):
    logits = jnp.dot(h_ref[...].astype(jnp.bfloat16), w_ref[...],
                     preferred_element_type=jnp.float32) + b_ref[...]
    e = jnp.exp(logits)
    fp = (e / jnp.sum(e, axis=1, keepdims=True)).astype(jnp.bfloat16)
    gid = bt_ref[0, 0, :]
    onehot = (gid[:, None] == lax.broadcasted_iota(
        jnp.int32, (_FIN_BLK, G), 1)).astype(jnp.bfloat16)
    return lax.dot_general(onehot, fp, (((0,), (0,)), ((), ())),
                           preferred_element_type=jnp.float32)


def _tc_fp1_body(h_ref, w_ref, b_ref, bt_ref, o_ref):
    i = pl.program_id(0)
    contrib = _fp_contrib(h_ref, w_ref, b_ref, bt_ref)

    @pl.when(i == 0)
    def _():
        o_ref[...] = contrib

    @pl.when(i > 0)
    def _():
        o_ref[...] += contrib


def _tc_fp2_body(acc_ref, h_ref, w_ref, b_ref, bt_ref, o_ref):
    i = pl.program_id(0)
    contrib = _fp_contrib(h_ref, w_ref, b_ref, bt_ref)

    @pl.when(i == 0)
    def _():
        o_ref[...] = acc_ref[...] + contrib

    @pl.when(i > 0)
    def _():
        o_ref[...] += contrib


_FIN_SPECS = [
    pl.BlockSpec((_FIN_BLK, D), lambda i: (i, 0)),
    pl.BlockSpec((D, FP), lambda i: (0, 0)),
    pl.BlockSpec((1, FP), lambda i: (0, 0)),
    pl.BlockSpec((1, 1, _FIN_BLK), lambda i: (i, 0, 0)),
]


def _tc_fp1(h, wt, b, batch3d):
    return pl.pallas_call(
        _tc_fp1_body,
        grid=(N // _FIN_BLK,),
        in_specs=_FIN_SPECS,
        out_specs=pl.BlockSpec((G, FP), lambda i: (0, 0)),
        out_shape=jax.ShapeDtypeStruct((G, FP), jnp.float32),
    )(h, wt, b, batch3d)


def _tc_fp2(acc, h, wt, b, batch3d):
    return pl.pallas_call(
        _tc_fp2_body,
        grid=(N // _FIN_BLK,),
        in_specs=[pl.BlockSpec((G, FP), lambda i: (0, 0))] + _FIN_SPECS,
        out_specs=pl.BlockSpec((G, FP), lambda i: (0, 0)),
        out_shape=jax.ShapeDtypeStruct((G, FP), jnp.float32),
    )(acc, h, wt, b, batch3d)


def kernel(x, edge_index, batch, H1_w, H1_b, W1_w, W1_b, H2_w, H2_b, W2_w,
           W2_b):
    src3d = edge_index[0].reshape(E // K, K)
    dst3d = edge_index[1].reshape(E // K, K)
    batch3d = batch.reshape(N // _FIN_BLK, 1, _FIN_BLK)
    w1t = W1_w.T.astype(jnp.bfloat16)
    w2t = W2_w.T.astype(jnp.bfloat16)

    p1 = _sc_agg(x, src3d, dst3d)
    h1 = _tc_layer(p1, x, H1_w.T, H1_b.reshape(1, D))
    p2 = _sc_agg(h1, src3d, dst3d)
    acc1 = _tc_fp1(h1, w1t, W1_b.reshape(1, FP), batch3d)
    h2 = _tc_layer(p2, h1, H2_w.T, H2_b.reshape(1, D))
    return _tc_fp2(acc1, h2, w2t, W2_b.reshape(1, FP), batch3d)


# speedup vs baseline: 3.3513x; 1.0884x over previous
import functools

import jax
import jax.numpy as jnp
from jax import lax
from jax.experimental import pallas as pl
from jax.experimental.pallas import tpu as pltpu
from jax.experimental.pallas import tpu_sc as plsc

N = 10000
E = 320000
D = 128
FP = 2048
G = 512

NC = 2
NS = 16
NW = NC * NS

K = 125
PER_TILE = E // NW
CH = PER_TILE // K
GC = 16
NG = CH // GC
ROWS_MAIN = 624
ROWS_TAIL = N - NS * ROWS_MAIN


def _sc_agg_body(table, src2d, dst2d, out, acc, src_g, dst_g, buf0, buf1,
                 g0s, g1s, i0s, i1s):
    c = lax.axis_index("c")
    s = lax.axis_index("s")
    wid = c * NS + s
    row0 = wid * CH
    bufs = (buf0, buf1)
    gsems = (g0s, g1s)
    isems = (i0s, i1s)

    pltpu.async_copy(src2d.at[pl.ds(row0, GC)], src_g.at[0], i0s)
    pltpu.async_copy(dst2d.at[pl.ds(row0, GC)], dst_g.at[0], i0s)
    pltpu.make_async_copy(src2d.at[pl.ds(row0, GC)], src_g.at[0],
                          i0s).wait()
    pltpu.make_async_copy(dst2d.at[pl.ds(row0, GC)], dst_g.at[0],
                          i0s).wait()
    pltpu.async_copy(table.at[src_g.at[0, 0]], buf0, g0s)
    pltpu.async_copy(src2d.at[pl.ds(row0 + GC, GC)], src_g.at[1], i1s)
    pltpu.async_copy(dst2d.at[pl.ds(row0 + GC, GC)], dst_g.at[1], i1s)

    zero = jnp.zeros((16,), jnp.float32)

    def zrow(r, carry):
        for cc in range(D // 16):
            buf1[r, pl.ds(cc * 16, 16)] = zero
        return carry

    lax.fori_loop(0, K, zrow, 0)
    base_row = s * ROWS_MAIN
    for kk in range(ROWS_MAIN // K):
        pltpu.sync_copy(buf1, acc.at[pl.ds(base_row + kk * K, K)])
    rem = ROWS_MAIN % K
    if rem:
        pltpu.sync_copy(
            buf1.at[pl.ds(0, rem)],
            acc.at[pl.ds(base_row + (ROWS_MAIN // K) * K, rem)])

    @pl.when(s == NS - 1)
    def _():
        pltpu.sync_copy(buf1.at[pl.ds(0, ROWS_TAIL)],
                        acc.at[pl.ds(NS * ROWS_MAIN, ROWS_TAIL)])

    plsc.subcore_barrier()
    pltpu.async_copy(table.at[src_g.at[0, 1]], buf1, g1s)

    for j in range(CH):
        b = j % 2
        slot = (j // GC) % 2
        row = j % GC
        pltpu.make_async_copy(table.at[src_g.at[0, 0]], bufs[b],
                              gsems[b]).wait()
        pltpu.sync_copy(bufs[b], acc.at[dst_g.at[slot, row]], add=True)
        jn = j + 2
        if jn < CH:
            slot_n = (jn // GC) % 2
            row_n = jn % GC
            if row_n == 0:
                pltpu.make_async_copy(src2d.at[pl.ds(row0, GC)],
                                      src_g.at[slot_n], isems[slot_n]).wait()
                pltpu.make_async_copy(dst2d.at[pl.ds(row0, GC)],
                                      dst_g.at[slot_n], isems[slot_n]).wait()
            pltpu.async_copy(table.at[src_g.at[slot_n, row_n]], bufs[b],
                             gsems[b])
        if row == GC - 1 and (j // GC) + 2 < NG:
            gbase = row0 + ((j // GC) + 2) * GC
            pltpu.async_copy(src2d.at[pl.ds(gbase, GC)], src_g.at[slot],
                             isems[slot])
            pltpu.async_copy(dst2d.at[pl.ds(gbase, GC)], dst_g.at[slot],
                             isems[slot])

    plsc.subcore_barrier()
    pltpu.sync_copy(acc.at[pl.ds(base_row, ROWS_MAIN)],
                    out.at[c, pl.ds(base_row, ROWS_MAIN)])

    @pl.when(s == NS - 1)
    def _():
        pltpu.sync_copy(acc.at[pl.ds(NS * ROWS_MAIN, ROWS_TAIL)],
                        out.at[c, pl.ds(NS * ROWS_MAIN, ROWS_TAIL)])


_sc_agg = functools.partial(
    pl.kernel,
    out_type=jax.ShapeDtypeStruct((NC, N, D), jnp.float32),
    cost_estimate=pl.CostEstimate(flops=85_000_000, transcendentals=0,
                                  bytes_accessed=200_000_000),
    mesh=plsc.VectorSubcoreMesh(core_axis_name="c", subcore_axis_name="s",
                                num_cores=NC, num_subcores=NS),
    scratch_types=[
        pltpu.VMEM_SHARED((N, D), jnp.float32),
        pltpu.VMEM((2, GC, K), jnp.int32),
        pltpu.VMEM((2, GC, K), jnp.int32),
        pltpu.VMEM((K, D), jnp.float32),
        pltpu.VMEM((K, D), jnp.float32),
        pltpu.SemaphoreType.DMA,
        pltpu.SemaphoreType.DMA,
        pltpu.SemaphoreType.DMA,
        pltpu.SemaphoreType.DMA,
    ],
)(_sc_agg_body)


def _tc_layer_body(p_ref, x_ref, w_ref, b_ref, o_ref):
    sm = p_ref[0] + p_ref[1] + x_ref[...]
    z = jnp.dot(sm, w_ref[...], preferred_element_type=jnp.float32)
    o_ref[...] = jax.nn.sigmoid(z + b_ref[...])


_LAYER_BLK = 2000


def _tc_layer(p, x, wt, b):
    nb = N // _LAYER_BLK
    return pl.pallas_call(
        _tc_layer_body,
        grid=(nb,),
        in_specs=[
            pl.BlockSpec((NC, _LAYER_BLK, D), lambda i: (0, i, 0)),
            pl.BlockSpec((_LAYER_BLK, D), lambda i: (i, 0)),
            pl.BlockSpec((D, D), lambda i: (0, 0)),
            pl.BlockSpec((1, D), lambda i: (0, 0)),
        ],
        out_specs=pl.BlockSpec((_LAYER_BLK, D), lambda i: (i, 0)),
        out_shape=jax.ShapeDtypeStruct((N, D), jnp.float32),
    )(p, x, wt, b)


_FIN_BLK = 400


SPAN = 64


def _fp_accum(h_ref, w_ref, b_ref, bt_ref, o_ref):
    logits = jnp.dot(h_ref[...].astype(jnp.bfloat16), w_ref[...],
                     preferred_element_type=jnp.float32) + b_ref[...]
    e = jnp.exp(logits)
    fp = (e / jnp.sum(e, axis=1, keepdims=True)).astype(jnp.bfloat16)
    gid = bt_ref[0, 0, :]
    g0 = jnp.minimum((jnp.min(gid) // 8) * 8, G - SPAN)
    fast = (jnp.max(gid) - g0) < SPAN

    @pl.when(fast)
    def _():
        onehot = ((gid - g0)[:, None] == lax.broadcasted_iota(
            jnp.int32, (_FIN_BLK, SPAN), 1)).astype(jnp.bfloat16)
        contrib = lax.dot_general(onehot, fp, (((0,), (0,)), ((), ())),
                                  preferred_element_type=jnp.float32)
        o_ref[pl.ds(g0, SPAN), :] += contrib

    @pl.when(jnp.logical_not(fast))
    def _():
        onehot = (gid[:, None] == lax.broadcasted_iota(
            jnp.int32, (_FIN_BLK, G), 1)).astype(jnp.bfloat16)
        contrib = lax.dot_general(onehot, fp, (((0,), (0,)), ((), ())),
                                  preferred_element_type=jnp.float32)
        o_ref[...] += contrib


def _tc_fp1_body(h_ref, w_ref, b_ref, bt_ref, o_ref):
    @pl.when(pl.program_id(0) == 0)
    def _():
        o_ref[...] = jnp.zeros((G, FP), jnp.float32)

    _fp_accum(h_ref, w_ref, b_ref, bt_ref, o_ref)


def _tc_fp2_body(acc_ref, h_ref, w_ref, b_ref, bt_ref, o_ref):
    @pl.when(pl.program_id(0) == 0)
    def _():
        o_ref[...] = acc_ref[...]

    _fp_accum(h_ref, w_ref, b_ref, bt_ref, o_ref)


_FIN_SPECS = [
    pl.BlockSpec((_FIN_BLK, D), lambda i: (i, 0)),
    pl.BlockSpec((D, FP), lambda i: (0, 0)),
    pl.BlockSpec((1, FP), lambda i: (0, 0)),
    pl.BlockSpec((1, 1, _FIN_BLK), lambda i: (i, 0, 0)),
]


def _tc_fp1(h, wt, b, batch3d):
    return pl.pallas_call(
        _tc_fp1_body,
        grid=(N // _FIN_BLK,),
        in_specs=_FIN_SPECS,
        out_specs=pl.BlockSpec((G, FP), lambda i: (0, 0)),
        out_shape=jax.ShapeDtypeStruct((G, FP), jnp.float32),
    )(h, wt, b, batch3d)


def _tc_fp2(acc, h, wt, b, batch3d):
    return pl.pallas_call(
        _tc_fp2_body,
        grid=(N // _FIN_BLK,),
        in_specs=[pl.BlockSpec((G, FP), lambda i: (0, 0))] + _FIN_SPECS,
        out_specs=pl.BlockSpec((G, FP), lambda i: (0, 0)),
        out_shape=jax.ShapeDtypeStruct((G, FP), jnp.float32),
    )(acc, h, wt, b, batch3d)


def kernel(x, edge_index, batch, H1_w, H1_b, W1_w, W1_b, H2_w, H2_b, W2_w,
           W2_b):
    src3d = edge_index[0].reshape(E // K, K)
    dst3d = edge_index[1].reshape(E // K, K)
    batch3d = batch.reshape(N // _FIN_BLK, 1, _FIN_BLK)
    w1t = W1_w.T.astype(jnp.bfloat16)
    w2t = W2_w.T.astype(jnp.bfloat16)

    p1 = _sc_agg(x, src3d, dst3d)
    h1 = _tc_layer(p1, x, H1_w.T, H1_b.reshape(1, D))
    p2 = _sc_agg(h1, src3d, dst3d)
    acc1 = _tc_fp1(h1, w1t, W1_b.reshape(1, FP), batch3d)
    h2 = _tc_layer(p2, h1, H2_w.T, H2_b.reshape(1, D))
    return _tc_fp2(acc1, h2, w2t, W2_b.reshape(1, FP), batch3d)


# speedup vs baseline: 3.4012x; 1.0149x over previous
import functools

import jax
import jax.numpy as jnp
from jax import lax
from jax.experimental import pallas as pl
from jax.experimental.pallas import tpu as pltpu
from jax.experimental.pallas import tpu_sc as plsc

N = 10000
E = 320000
D = 128
FP = 2048
G = 512

NC = 2
NS = 16
NW = NC * NS

K = 125
PER_TILE = E // NW
CH = PER_TILE // K
GC = 16
NG = CH // GC
ROWS_MAIN = 624
ROWS_TAIL = N - NS * ROWS_MAIN


def _sc_agg_body(table, src2d, dst2d, out, acc, src_g, dst_g, buf0, buf1,
                 g0s, g1s, i0s, i1s):
    c = lax.axis_index("c")
    s = lax.axis_index("s")
    wid = c * NS + s
    row0 = wid * CH
    bufs = (buf0, buf1)
    gsems = (g0s, g1s)
    isems = (i0s, i1s)

    pltpu.async_copy(src2d.at[pl.ds(row0, GC)], src_g.at[0], i0s)
    pltpu.async_copy(dst2d.at[pl.ds(row0, GC)], dst_g.at[0], i0s)
    pltpu.make_async_copy(src2d.at[pl.ds(row0, GC)], src_g.at[0],
                          i0s).wait()
    pltpu.make_async_copy(dst2d.at[pl.ds(row0, GC)], dst_g.at[0],
                          i0s).wait()
    pltpu.async_copy(table.at[src_g.at[0, 0]], buf0, g0s)
    pltpu.async_copy(src2d.at[pl.ds(row0 + GC, GC)], src_g.at[1], i1s)
    pltpu.async_copy(dst2d.at[pl.ds(row0 + GC, GC)], dst_g.at[1], i1s)

    zero = jnp.zeros((16,), jnp.float32)

    def zrow(r, carry):
        for cc in range(D // 16):
            buf1[r, pl.ds(cc * 16, 16)] = zero
        return carry

    lax.fori_loop(0, K, zrow, 0)
    base_row = s * ROWS_MAIN
    for kk in range(ROWS_MAIN // K):
        pltpu.sync_copy(buf1, acc.at[pl.ds(base_row + kk * K, K)])
    rem = ROWS_MAIN % K
    if rem:
        pltpu.sync_copy(
            buf1.at[pl.ds(0, rem)],
            acc.at[pl.ds(base_row + (ROWS_MAIN // K) * K, rem)])

    @pl.when(s == NS - 1)
    def _():
        pltpu.sync_copy(buf1.at[pl.ds(0, ROWS_TAIL)],
                        acc.at[pl.ds(NS * ROWS_MAIN, ROWS_TAIL)])

    plsc.subcore_barrier()
    pltpu.async_copy(table.at[src_g.at[0, 1]], buf1, g1s)

    for j in range(CH):
        b = j % 2
        slot = (j // GC) % 2
        row = j % GC
        pltpu.make_async_copy(table.at[src_g.at[0, 0]], bufs[b],
                              gsems[b]).wait()
        pltpu.sync_copy(bufs[b], acc.at[dst_g.at[slot, row]], add=True)
        jn = j + 2
        if jn < CH:
            slot_n = (jn // GC) % 2
            row_n = jn % GC
            if row_n == 0:
                pltpu.make_async_copy(src2d.at[pl.ds(row0, GC)],
                                      src_g.at[slot_n], isems[slot_n]).wait()
                pltpu.make_async_copy(dst2d.at[pl.ds(row0, GC)],
                                      dst_g.at[slot_n], isems[slot_n]).wait()
            pltpu.async_copy(table.at[src_g.at[slot_n, row_n]], bufs[b],
                             gsems[b])
        if row == GC - 1 and (j // GC) + 2 < NG:
            gbase = row0 + ((j // GC) + 2) * GC
            pltpu.async_copy(src2d.at[pl.ds(gbase, GC)], src_g.at[slot],
                             isems[slot])
            pltpu.async_copy(dst2d.at[pl.ds(gbase, GC)], dst_g.at[slot],
                             isems[slot])

    plsc.subcore_barrier()
    pltpu.sync_copy(acc.at[pl.ds(base_row, ROWS_MAIN)],
                    out.at[c, pl.ds(base_row, ROWS_MAIN)])

    @pl.when(s == NS - 1)
    def _():
        pltpu.sync_copy(acc.at[pl.ds(NS * ROWS_MAIN, ROWS_TAIL)],
                        out.at[c, pl.ds(NS * ROWS_MAIN, ROWS_TAIL)])


_sc_agg = functools.partial(
    pl.kernel,
    out_type=jax.ShapeDtypeStruct((NC, N, D), jnp.float32),
    cost_estimate=pl.CostEstimate(flops=85_000_000, transcendentals=0,
                                  bytes_accessed=200_000_000),
    mesh=plsc.VectorSubcoreMesh(core_axis_name="c", subcore_axis_name="s",
                                num_cores=NC, num_subcores=NS),
    scratch_types=[
        pltpu.VMEM_SHARED((N, D), jnp.float32),
        pltpu.VMEM((2, GC, K), jnp.int32),
        pltpu.VMEM((2, GC, K), jnp.int32),
        pltpu.VMEM((K, D), jnp.float32),
        pltpu.VMEM((K, D), jnp.float32),
        pltpu.SemaphoreType.DMA,
        pltpu.SemaphoreType.DMA,
        pltpu.SemaphoreType.DMA,
        pltpu.SemaphoreType.DMA,
    ],
)(_sc_agg_body)


def _tc_layer_body(p_ref, x_ref, w_ref, b_ref, o_ref):
    sm = p_ref[0] + p_ref[1] + x_ref[...]
    z = jnp.dot(sm, w_ref[...], preferred_element_type=jnp.float32)
    o_ref[...] = jax.nn.sigmoid(z + b_ref[...])


_LAYER_BLK = 2000


def _tc_layer(p, x, wt, b):
    nb = N // _LAYER_BLK
    return pl.pallas_call(
        _tc_layer_body,
        grid=(nb,),
        in_specs=[
            pl.BlockSpec((NC, _LAYER_BLK, D), lambda i: (0, i, 0)),
            pl.BlockSpec((_LAYER_BLK, D), lambda i: (i, 0)),
            pl.BlockSpec((D, D), lambda i: (0, 0)),
            pl.BlockSpec((1, D), lambda i: (0, 0)),
        ],
        out_specs=pl.BlockSpec((_LAYER_BLK, D), lambda i: (i, 0)),
        out_shape=jax.ShapeDtypeStruct((N, D), jnp.float32),
    )(p, x, wt, b)


_FIN_BLK = 400


SPAN = 64


def _fp_accum(h_ref, w_ref, b_ref, bt_ref, o_ref):
    logits = jnp.dot(h_ref[...].astype(jnp.bfloat16), w_ref[...],
                     preferred_element_type=jnp.float32) + b_ref[...]
    e = jnp.exp(logits)
    fp = (e / jnp.sum(e, axis=1, keepdims=True)).astype(jnp.bfloat16)
    gid = bt_ref[0, 0, :]
    g0 = jnp.minimum((jnp.min(gid) // 8) * 8, G - SPAN)
    fast = (jnp.max(gid) - g0) < SPAN

    @pl.when(fast)
    def _():
        onehot = ((gid - g0)[:, None] == lax.broadcasted_iota(
            jnp.int32, (_FIN_BLK, SPAN), 1)).astype(jnp.bfloat16)
        contrib = lax.dot_general(onehot, fp, (((0,), (0,)), ((), ())),
                                  preferred_element_type=jnp.float32)
        o_ref[pl.ds(g0, SPAN), :] += contrib

    @pl.when(jnp.logical_not(fast))
    def _():
        onehot = (gid[:, None] == lax.broadcasted_iota(
            jnp.int32, (_FIN_BLK, G), 1)).astype(jnp.bfloat16)
        contrib = lax.dot_general(onehot, fp, (((0,), (0,)), ((), ())),
                                  preferred_element_type=jnp.float32)
        o_ref[...] += contrib


def _tc_fp1_body(h_ref, w_ref, b_ref, bt_ref, o_ref):
    @pl.when(pl.program_id(0) == 0)
    def _():
        o_ref[...] = jnp.zeros((G, FP), jnp.float32)

    _fp_accum(h_ref, w_ref, b_ref, bt_ref, o_ref)


def _tc_fp2_body(acc_ref, p_ref, h1_ref, hw_ref, hb_ref, w_ref, b_ref,
                 bt_ref, o_ref, h2_scr):
    @pl.when(pl.program_id(0) == 0)
    def _():
        o_ref[...] = acc_ref[...]

    sm = p_ref[0] + p_ref[1] + h1_ref[...]
    z = jnp.dot(sm, hw_ref[...], preferred_element_type=jnp.float32)
    h2_scr[...] = jax.nn.sigmoid(z + hb_ref[...])
    _fp_accum(h2_scr, w_ref, b_ref, bt_ref, o_ref)


_FIN_SPECS = [
    pl.BlockSpec((_FIN_BLK, D), lambda i: (i, 0)),
    pl.BlockSpec((D, FP), lambda i: (0, 0)),
    pl.BlockSpec((1, FP), lambda i: (0, 0)),
    pl.BlockSpec((1, 1, _FIN_BLK), lambda i: (i, 0, 0)),
]


def _tc_fp1(h, wt, b, batch3d):
    return pl.pallas_call(
        _tc_fp1_body,
        grid=(N // _FIN_BLK,),
        in_specs=_FIN_SPECS,
        out_specs=pl.BlockSpec((G, FP), lambda i: (0, 0)),
        out_shape=jax.ShapeDtypeStruct((G, FP), jnp.float32),
    )(h, wt, b, batch3d)


def _tc_fp2(acc, p2, h1, hwt, hb, wt, b, batch3d):
    return pl.pallas_call(
        _tc_fp2_body,
        grid=(N // _FIN_BLK,),
        in_specs=[
            pl.BlockSpec((G, FP), lambda i: (0, 0)),
            pl.BlockSpec((NC, _FIN_BLK, D), lambda i: (0, i, 0)),
            pl.BlockSpec((_FIN_BLK, D), lambda i: (i, 0)),
            pl.BlockSpec((D, D), lambda i: (0, 0)),
            pl.BlockSpec((1, D), lambda i: (0, 0)),
        ] + _FIN_SPECS[1:],
        out_specs=pl.BlockSpec((G, FP), lambda i: (0, 0)),
        out_shape=jax.ShapeDtypeStruct((G, FP), jnp.float32),
        scratch_shapes=[pltpu.VMEM((_FIN_BLK, D), jnp.float32)],
    )(acc, p2, h1, hwt, hb, wt, b, batch3d)


def kernel(x, edge_index, batch, H1_w, H1_b, W1_w, W1_b, H2_w, H2_b, W2_w,
           W2_b):
    src3d = edge_index[0].reshape(E // K, K)
    dst3d = edge_index[1].reshape(E // K, K)
    batch3d = batch.reshape(N // _FIN_BLK, 1, _FIN_BLK)
    w1t = W1_w.T.astype(jnp.bfloat16)
    w2t = W2_w.T.astype(jnp.bfloat16)

    p1 = _sc_agg(x, src3d, dst3d)
    h1 = _tc_layer(p1, x, H1_w.T, H1_b.reshape(1, D))
    acc1 = _tc_fp1(h1, w1t, W1_b.reshape(1, FP), batch3d)
    p2 = _sc_agg(h1, src3d, dst3d)
    return _tc_fp2(acc1, p2, h1, H2_w.T, H2_b.reshape(1, D),
                   w2t, W2_b.reshape(1, FP), batch3d)
